# Initial kernel scaffold; baseline (speedup 1.0000x reference)
import jax
import jax.numpy as jnp
from jax.experimental import pallas as pl


def kernel(x, edge_index, batch_index, W1, b1, bn_w, bn_b, W2, b2, Wp, bp):
    raise NotImplementedError("write your pallas kernel here")


# speedup vs baseline: 13.8803x; 13.8803x over previous
import functools

import jax
import jax.numpy as jnp
from jax import lax
from jax.experimental import pallas as pl
from jax.experimental.pallas import tpu as pltpu
from jax.experimental.pallas import tpu_sc as plsc

_N = 10000
_D = 128
_H = 128
_E = 320000
_G = 64

_NC = 2
_NS = 16
_NW = _NC * _NS
_EPW = _E // _NW
_K = 80
_NCHUNK = _EPW // _K
_HH = _H // 2
_EPS = _E // _NS
_NCH2 = _EPS // _K

_RPT8 = 624
_TAIL = _N - _NS * _RPT8

_BN = 1000
_NB = _N // _BN

_mesh = plsc.VectorSubcoreMesh(core_axis_name="c", subcore_axis_name="s")


@functools.partial(
    pl.kernel,
    out_type=jax.ShapeDtypeStruct((_NC, _N, 16), jnp.float32),
    mesh=_mesh,
    scratch_types=[
        pltpu.VMEM((_NCHUNK, _K), jnp.int32),
        pltpu.VMEM((_K, 16), jnp.float32),
        pltpu.VMEM((_RPT8, 16), jnp.float32),
        pltpu.VMEM_SHARED((_N, 16), jnp.float32),
    ],
    compiler_params=pltpu.CompilerParams(use_tc_tiling_on_sc=False),
)
def _deg_kernel(dst_hbm, deg_out, idx_v, ones_v, zero_v, deg_sp):
    c = lax.axis_index("c")
    s = lax.axis_index("s")

    def fill_ones(i, carry):
        ones_v[i, :] = jnp.full((16,), 1.0, jnp.float32)
        return carry

    lax.fori_loop(0, _K, fill_ones, 0)

    def fill_zero(i, carry):
        zero_v[i, :] = jnp.zeros((16,), jnp.float32)
        return carry

    lax.fori_loop(0, _RPT8, fill_zero, 0)

    pltpu.sync_copy(zero_v, deg_sp.at[pl.ds(s * _RPT8, _RPT8)])

    @pl.when(s == _NS - 1)
    def _():
        pltpu.sync_copy(zero_v.at[pl.ds(0, _TAIL)],
                        deg_sp.at[pl.ds(_NS * _RPT8, _TAIL)])

    plsc.subcore_barrier()

    pltpu.sync_copy(dst_hbm.at[c, s], idx_v)

    def chunk(j, carry):
        pltpu.sync_copy(ones_v, deg_sp.at[idx_v.at[j]], add=True)
        return carry

    lax.fori_loop(0, _NCHUNK, chunk, 0)
    plsc.subcore_barrier()

    pltpu.sync_copy(deg_sp.at[pl.ds(s * _RPT8, _RPT8)],
                    deg_out.at[c, pl.ds(s * _RPT8, _RPT8)])

    @pl.when(s == _NS - 1)
    def _():
        pltpu.sync_copy(deg_sp.at[pl.ds(_NS * _RPT8, _TAIL)],
                        deg_out.at[c, pl.ds(_NS * _RPT8, _TAIL)])


@functools.partial(
    pl.kernel,
    out_type=jax.ShapeDtypeStruct((_NC, _N, _HH), jnp.float32),
    mesh=_mesh,
    scratch_types=[
        pltpu.VMEM((_NCH2, _K), jnp.int32),
        pltpu.VMEM((_NCH2, _K), jnp.int32),
        pltpu.VMEM((_K, _HH), jnp.float32),
        pltpu.VMEM((208, _HH), jnp.float32),
        pltpu.VMEM_SHARED((_N, _HH), jnp.float32),
        pltpu.SemaphoreType.DMA,
    ],
    compiler_params=pltpu.CompilerParams(use_tc_tiling_on_sc=False),
)
def _agg_kernel(ya_hbm, yb_hbm, src_hbm, dst_hbm, out_hbm,
                src_v, dst_v, rows_v, zero_v, acc_sp, sem):
    c = lax.axis_index("c")
    s = lax.axis_index("s")

    def fill_zero(i, carry):
        for l in range(_HH // 16):
            zero_v[i, pl.ds(l * 16, 16)] = jnp.zeros((16,), jnp.float32)
        return carry

    lax.fori_loop(0, 208, fill_zero, 0)
    for t in range(_RPT8 // 208):
        pltpu.sync_copy(zero_v, acc_sp.at[pl.ds(s * _RPT8 + t * 208, 208)])

    @pl.when(s == _NS - 1)
    def _():
        pltpu.sync_copy(zero_v.at[pl.ds(0, _TAIL)],
                        acc_sp.at[pl.ds(_NS * _RPT8, _TAIL)])

    plsc.subcore_barrier()

    pltpu.sync_copy(src_hbm.at[s], src_v)
    pltpu.sync_copy(dst_hbm.at[s], dst_v)

    def run_edges(y_hbm):
        def chunk(j, carry):
            pltpu.async_copy(y_hbm.at[src_v.at[j]], rows_v, sem).wait()
            pltpu.sync_copy(rows_v, acc_sp.at[dst_v.at[j]], add=True)
            return carry

        lax.fori_loop(0, _NCH2, chunk, 0)

    @pl.when(c == 0)
    def _():
        run_edges(ya_hbm)

    @pl.when(c == 1)
    def _():
        run_edges(yb_hbm)
    plsc.subcore_barrier()

    pltpu.sync_copy(acc_sp.at[pl.ds(s * _RPT8, _RPT8)],
                    out_hbm.at[c, pl.ds(s * _RPT8, _RPT8)])

    @pl.when(s == _NS - 1)
    def _():
        pltpu.sync_copy(acc_sp.at[pl.ds(_NS * _RPT8, _TAIL)],
                        out_hbm.at[c, pl.ds(_NS * _RPT8, _TAIL)])


def _dinv_of(deg_ref):
    deg = deg_ref[0, :, 0:1] + deg_ref[1, :, 0:1] + 1.0
    return lax.rsqrt(deg)


def _k1_body(deg_ref, x_ref, w_ref, ya_ref, yb_ref):
    dinv = _dinv_of(deg_ref)
    xw = jnp.dot(x_ref[...], w_ref[...], preferred_element_type=jnp.float32)
    y = xw * dinv
    ya_ref[...] = y[:, :_HH]
    yb_ref[...] = y[:, _HH:]


def _halves(ref):
    return jnp.concatenate([ref[0], ref[1]], axis=-1)


def _k3_body(agg_ref, ya_ref, yb_ref, deg_ref, b_ref, h_ref, s_ref, q_ref):
    i = pl.program_id(0)
    dinv = _dinv_of(deg_ref)
    y = jnp.concatenate([ya_ref[...], yb_ref[...]], axis=-1)
    h = dinv * (_halves(agg_ref) + y) + b_ref[...]
    h_ref[...] = h
    cs = jnp.sum(h, axis=0, keepdims=True)
    cq = jnp.sum(h * h, axis=0, keepdims=True)

    @pl.when(i == 0)
    def _():
        s_ref[...] = cs
        q_ref[...] = cq

    @pl.when(i > 0)
    def _():
        s_ref[...] = s_ref[...] + cs
        q_ref[...] = q_ref[...] + cq


_SELU_ALPHA = 1.6732632423543772
_SELU_SCALE = 1.0507009873554805


def _k4_body(h_ref, s_ref, q_ref, bnw_ref, bnb_ref, w2_ref, deg_ref,
             ya_ref, yb_ref):
    m = s_ref[...] * (1.0 / _N)
    v = q_ref[...] * (1.0 / _N) - m * m
    scale = lax.rsqrt(v + 1e-5) * bnw_ref[...]
    hn = (h_ref[...] - m) * scale + bnb_ref[...]
    neg = _SELU_ALPHA * (jnp.exp(jnp.minimum(hn, 0.0)) - 1.0)
    hs = _SELU_SCALE * jnp.where(hn > 0, hn, neg)
    dinv = _dinv_of(deg_ref)
    y2 = jnp.dot(hs, w2_ref[...], preferred_element_type=jnp.float32) * dinv
    ya_ref[...] = y2[:, :_HH]
    yb_ref[...] = y2[:, _HH:]


def _k6_body(agg_ref, ya_ref, yb_ref, deg_ref, b_ref, bi_ref, wp_ref, bp_ref,
             o_ref, sum_s, cnt_s, max_s):
    i = pl.program_id(0)
    dinv = _dinv_of(deg_ref)
    y = jnp.concatenate([ya_ref[...], yb_ref[...]], axis=-1)
    h = dinv * (_halves(agg_ref) + y) + b_ref[...]

    bi = bi_ref[0, 0, :]
    gid = lax.broadcasted_iota(jnp.int32, (_G, _BN), 0)
    oh = (gid == bi[None, :]).astype(jnp.float32)
    ps = jnp.dot(oh, h, preferred_element_type=jnp.float32)
    pc = jnp.broadcast_to(jnp.sum(oh, axis=1, keepdims=True), (_G, _H))

    neg = jnp.float32(-3.0e38)
    rows = []
    for g in range(_G):
        mg = jnp.max(jnp.where(bi[:, None] == g, h, neg), axis=0,
                     keepdims=True)
        rows.append(mg)
    bm = jnp.concatenate(rows, axis=0)

    @pl.when(i == 0)
    def _():
        sum_s[...] = ps
        cnt_s[...] = pc
        max_s[...] = bm

    @pl.when(i > 0)
    def _():
        sum_s[...] = sum_s[...] + ps
        cnt_s[...] = cnt_s[...] + pc
        max_s[...] = jnp.maximum(max_s[...], bm)

    @pl.when(i == _NB - 1)
    def _():
        ssum = sum_s[...]
        cnt = cnt_s[...]
        mean = ssum / jnp.maximum(cnt, 1.0)
        mx = jnp.where(cnt > 0, max_s[...], 0.0)
        o_ref[...] = (jnp.dot(ssum, wp_ref[0], preferred_element_type=jnp.float32)
                      + jnp.dot(mean, wp_ref[1], preferred_element_type=jnp.float32)
                      + jnp.dot(mx, wp_ref[2], preferred_element_type=jnp.float32)
                      + bp_ref[...])


_deg_spec = pl.BlockSpec((_NC, _BN, 16), lambda i: (0, i, 0))
_row_spec = pl.BlockSpec((_BN, _H), lambda i: (i, 0))
_halfrow_spec = pl.BlockSpec((_BN, _HH), lambda i: (i, 0))
_half_spec = pl.BlockSpec((2, _BN, _HH), lambda i: (0, i, 0))
_vec_spec = pl.BlockSpec((1, _H), lambda i: (0, 0))
_w_spec = pl.BlockSpec((_H, _H), lambda i: (0, 0))

_yhalf_shape = [jax.ShapeDtypeStruct((_N, _HH), jnp.float32),
                jax.ShapeDtypeStruct((_N, _HH), jnp.float32)]


def _k1(deg, x, w1):
    return pl.pallas_call(
        _k1_body,
        grid=(_NB,),
        in_specs=[_deg_spec, _row_spec, _w_spec],
        out_specs=[_halfrow_spec, _halfrow_spec],
        out_shape=_yhalf_shape,
    )(deg, x, w1)


def _k3(agg, ya, yb, deg, b):
    return pl.pallas_call(
        _k3_body,
        grid=(_NB,),
        in_specs=[_half_spec, _halfrow_spec, _halfrow_spec, _deg_spec,
                  _vec_spec],
        out_specs=[_row_spec, _vec_spec, _vec_spec],
        out_shape=[
            jax.ShapeDtypeStruct((_N, _H), jnp.float32),
            jax.ShapeDtypeStruct((1, _H), jnp.float32),
            jax.ShapeDtypeStruct((1, _H), jnp.float32),
        ],
    )(agg, ya, yb, deg, b)


def _k4(h, s, q, bnw, bnb, w2, deg):
    return pl.pallas_call(
        _k4_body,
        grid=(_NB,),
        in_specs=[_row_spec, _vec_spec, _vec_spec, _vec_spec, _vec_spec,
                  _w_spec, _deg_spec],
        out_specs=[_halfrow_spec, _halfrow_spec],
        out_shape=_yhalf_shape,
    )(h, s, q, bnw, bnb, w2, deg)


def _k6(agg, ya, yb, deg, b, bi3, wp3, bp):
    return pl.pallas_call(
        _k6_body,
        grid=(_NB,),
        in_specs=[_half_spec, _halfrow_spec, _halfrow_spec, _deg_spec,
                  _vec_spec,
                  pl.BlockSpec((1, 1, _BN), lambda i: (i, 0, 0)),
                  pl.BlockSpec((3, _H, _H), lambda i: (0, 0, 0)),
                  _vec_spec],
        out_specs=pl.BlockSpec((_G, _H), lambda i: (0, 0)),
        out_shape=jax.ShapeDtypeStruct((_G, _H), jnp.float32),
        scratch_shapes=[
            pltpu.VMEM((_G, _H), jnp.float32),
            pltpu.VMEM((_G, _H), jnp.float32),
            pltpu.VMEM((_G, _H), jnp.float32),
        ],
    )(agg, ya, yb, deg, b, bi3, wp3, bp)


def kernel(x, edge_index, batch_index, W1, b1, bn_w, bn_b, W2, b2, Wp, bp):
    srcd = edge_index[0].reshape(_NC, _NS, _NCHUNK, _K)
    dstd = edge_index[1].reshape(_NC, _NS, _NCHUNK, _K)
    src = edge_index[0].reshape(_NS, _NCH2, _K)
    dst = edge_index[1].reshape(_NS, _NCH2, _K)
    bi3 = batch_index.reshape(_NB, 1, _BN)
    wp3 = Wp.reshape(3, _H, _H)
    b1r = b1.reshape(1, _H)
    b2r = b2.reshape(1, _H)
    bnwr = bn_w.reshape(1, _H)
    bnbr = bn_b.reshape(1, _H)
    bpr = bp.reshape(1, _H)

    deg = _deg_kernel(dstd)
    y1a, y1b = _k1(deg, x, W1)
    agg1 = _agg_kernel(y1a, y1b, src, dst)
    h1, s1, q1 = _k3(agg1, y1a, y1b, deg, b1r)
    y2a, y2b = _k4(h1, s1, q1, bnwr, bnbr, W2, deg)
    agg2 = _agg_kernel(y2a, y2b, src, dst)
    return _k6(agg2, y2a, y2b, deg, b2r, bi3, wp3, bpr)


def _mk_spmem_probe(rows, tag):
    @functools.partial(
        pl.kernel,
        out_type=jax.ShapeDtypeStruct((_NC, 8, 128), jnp.float32),
        mesh=_mesh,
        scratch_types=[
            pltpu.VMEM((8, 128), jnp.float32),
            pltpu.VMEM_SHARED((rows, 128), jnp.float32),
        ],
    )
    def _probe(x_hbm, o_hbm, v, sp):
        c = lax.axis_index("c")
        s = lax.axis_index("s")

        @pl.when(s == 0)
        def _():
            pltpu.sync_copy(x_hbm, v)
            pltpu.sync_copy(v, sp.at[pl.ds(0, 8)])
            pltpu.sync_copy(sp.at[pl.ds(0, 8)], o_hbm.at[c])

    return _probe


def probe_fn(x, edge_index, batch_index, W1, b1, bn_w, bn_b, W2, b2, Wp, bp):
    import os as _os
    rows = int(_os.environ.get("PROBE_ROWS", "5000"))
    ninst = int(_os.environ.get("PROBE_NINST", "1"))
    x8 = x[:8]
    outs = []
    for t in range(ninst):
        outs.append(_mk_spmem_probe(rows, t)(x8 + t))
    return sum(o.sum() for o in outs)
